# baseline (speedup 1.0000x reference)
import functools

import jax
import jax.numpy as jnp
from jax import lax
from jax.experimental import pallas as pl
from jax.experimental.pallas import tpu as pltpu
from jax.experimental.pallas import tpu_sc as plsc

_BETA = 0.25


def _bf16r(x):
    return x.astype(jnp.bfloat16).astype(jnp.float32)


def _argmin_body(statics, z_ref, cbT_ref, a_ref, b_ref, col_ref,
                 idx_ref, loss_ref, cv, ci, wv, wi, vb, acc):
    nj, bn, scale, t1, off1, t2, off2 = statics
    i = pl.program_id(0)
    j = pl.program_id(1)
    bm = cv.shape[0]
    inf = jnp.float32(jnp.inf)

    @pl.when(j == 0)
    def _init():
        cv[...] = jnp.full(cv.shape, inf, jnp.float32)
        ci[...] = jnp.zeros(ci.shape, jnp.int32)
        wv[...] = jnp.full(wv.shape, inf, jnp.float32)
        wi[...] = jnp.zeros(wi.shape, jnp.int32)
        vb[...] = jnp.zeros(vb.shape, jnp.float32)

    dot2 = lax.dot_general(z_ref[0], cbT_ref[...], (((0,), (0,)), ((), ())),
                           preferred_element_type=jnp.float32)
    scores = (a_ref[...] + b_ref[...]) - dot2
    colf = col_ref[...]

    def lexmin(s):
        m = jnp.min(s, axis=1, keepdims=True)
        gf = jnp.min(jnp.where(s == m, colf, jnp.float32(1 << 24)),
                     axis=1, keepdims=True)
        return m, gf.astype(jnp.int32)

    def merge_window(m, g):
        upd = m < wv[...]
        wv[...] = jnp.where(upd, m, wv[...])
        wi[...] = jnp.where(upd, g, wi[...])

    def close_window():
        take = wv[...] < cv[...]
        cv[...] = jnp.where(take, _bf16r(wv[...]), cv[...])
        ci[...] = jnp.where(take, wi[...], ci[...])
        vb[...] = jnp.where(take, wv[...], vb[...])

    @pl.when(jnp.logical_and(j != t1, j != t2))
    def _whole_tile():
        m, g = lexmin(scores)
        merge_window(m, g)

    def _split_tile(bound):
        cut = colf < jnp.float32(bound)
        seg1 = jnp.where(cut, scores, inf)
        m1, g1 = lexmin(seg1)
        merge_window(m1, g1)
        close_window()
        seg2 = jnp.where(cut, inf, scores)
        m2, g2 = lexmin(seg2)
        wv[...] = m2
        wi[...] = g2

    @pl.when(j == t1)
    def _boundary1():
        _split_tile((t1 * bn) + off1)

    @pl.when(j == t2)
    def _boundary2():
        _split_tile((t2 * bn) + off2)

    @pl.when(j == nj - 1)
    def _finish():
        take = wv[...] < cv[...]
        fidx = jnp.where(take, wi[...], ci[...])
        fval = jnp.where(take, wv[...], vb[...])
        idx_ref[...] = fidx

        @pl.when(i == 0)
        def _zero():
            acc[0] = 0.0

        acc[0] = acc[0] + jnp.sum(fval)

        @pl.when(i == pl.num_programs(0) - 1)
        def _emit():
            loss_ref[0, 0] = acc[0] * scale


def _distance_argmin(z16, cbT16, a, b2, colf, bm, bn):
    nb, e_dim, hw = z16.shape
    n_tok = nb * hw
    n_e = cbT16.shape[1]
    per_b = hw // bm
    grid = (n_tok // bm, n_e // bn)
    scale = (1.0 + _BETA) / (n_tok * e_dim)
    w = ((-(-n_e // 3) + 7) // 8) * 8
    t1, off1 = w // bn, w % bn
    t2, off2 = (2 * w) // bn, (2 * w) % bn
    statics = (grid[1], bn, scale, t1, off1, t2, off2)
    body = functools.partial(_argmin_body, statics)
    return pl.pallas_call(
        body,
        grid=grid,
        in_specs=[
            pl.BlockSpec((1, e_dim, bm),
                         lambda i, j: (i // per_b, 0, i % per_b)),
            pl.BlockSpec((e_dim, bn), lambda i, j: (0, j)),
            pl.BlockSpec((bm, 1), lambda i, j: (i, 0)),
            pl.BlockSpec((1, bn), lambda i, j: (0, j)),
            pl.BlockSpec((1, bn), lambda i, j: (0, j)),
        ],
        out_specs=[
            pl.BlockSpec((bm, 1), lambda i, j: (i, 0)),
            pl.BlockSpec(memory_space=pltpu.SMEM),
        ],
        out_shape=[
            jax.ShapeDtypeStruct((n_tok, 1), jnp.int32),
            jax.ShapeDtypeStruct((1, 1), jnp.float32),
        ],
        scratch_shapes=[
            pltpu.VMEM((bm, 1), jnp.float32),
            pltpu.VMEM((bm, 1), jnp.int32),
            pltpu.VMEM((bm, 1), jnp.float32),
            pltpu.VMEM((bm, 1), jnp.int32),
            pltpu.VMEM((bm, 1), jnp.float32),
            pltpu.SMEM((1,), jnp.float32),
        ],
        compiler_params=pltpu.CompilerParams(
            dimension_semantics=("arbitrary", "arbitrary")),
    )(z16, cbT16, a, b2, colf)


def _sc_gather(codebook, idx):
    n_e, e_dim = codebook.shape
    n_tok = idx.shape[0]
    info = plsc.get_sparse_core_info()
    nc, ns = info.num_cores, info.num_subcores
    nw = nc * ns
    b_per_w = n_tok // nw
    ch = 128
    n_ch = b_per_w // ch
    mesh = plsc.VectorSubcoreMesh(core_axis_name="c", subcore_axis_name="s")

    @functools.partial(
        pl.kernel, mesh=mesh,
        out_type=jax.ShapeDtypeStruct((n_tok, e_dim), jnp.float32),
        scratch_types=[
            pltpu.VMEM((b_per_w,), jnp.int32),
            pltpu.VMEM((ch, e_dim), jnp.float32),
            pltpu.SemaphoreType.DMA,
        ],
    )
    def gathered(table_hbm, idx_hbm, out_hbm, idx_v, rows_v, sem):
        wid = lax.axis_index("s") * nc + lax.axis_index("c")
        base = wid * b_per_w
        pltpu.sync_copy(idx_hbm.at[pl.ds(base, b_per_w)], idx_v)
        for c in range(n_ch):
            pltpu.async_copy(
                table_hbm.at[idx_v.at[pl.ds(c * ch, ch)]], rows_v, sem).wait()
            pltpu.sync_copy(rows_v, out_hbm.at[pl.ds(base + c * ch, ch)])

    return gathered(codebook, idx)


def kernel(z, codebook):
    B, C, H, W = z.shape
    n_e, e_dim = codebook.shape
    n_tok = B * H * W

    a = jnp.sum(z ** 2, axis=1).reshape(n_tok, 1)
    b2 = jnp.sum(codebook ** 2, axis=1).reshape(1, n_e)
    z16 = (z * 2.0).astype(jnp.bfloat16).reshape(B, C, H * W)
    cbT16 = codebook.T.astype(jnp.bfloat16)
    colf = jnp.arange(n_e, dtype=jnp.float32).reshape(1, n_e)

    bm = min(2048, H * W)
    bn = min(4096, n_e)
    idx2, loss11 = _distance_argmin(z16, cbT16, a, b2, colf, bm, bn)
    idx = idx2.reshape(n_tok)

    z_q = _sc_gather(codebook, idx)

    z_q_out = jnp.transpose(z_q.reshape(B, H, W, C), (0, 3, 1, 2))
    idx_out = idx.reshape(B, 1, H, W)
    return (z_q_out, loss11[0, 0], idx_out)


# speedup vs baseline: 1.1829x; 1.1829x over previous
import functools

import jax
import jax.numpy as jnp
from jax import lax
from jax.experimental import pallas as pl
from jax.experimental.pallas import tpu as pltpu
from jax.experimental.pallas import tpu_sc as plsc

_BETA = 0.25


def _bf16r(x):
    return x.astype(jnp.bfloat16).astype(jnp.float32)


def _argmin_body(statics, z_ref, cbT_ref, a_ref, b_ref, col_ref,
                 idx_ref, loss_ref, cv, ci, wv, wi, vb, acc):
    nj, bn, scale, t1, off1, t2, off2 = statics
    i = pl.program_id(0)
    j = pl.program_id(1)
    bm = cv.shape[0]
    inf = jnp.float32(jnp.inf)

    @pl.when(j == 0)
    def _init():
        cv[...] = jnp.full(cv.shape, inf, jnp.float32)
        ci[...] = jnp.zeros(ci.shape, jnp.int32)
        wv[...] = jnp.full(wv.shape, inf, jnp.float32)
        wi[...] = jnp.zeros(wi.shape, jnp.int32)
        vb[...] = jnp.zeros(vb.shape, jnp.float32)

    dot2 = lax.dot_general(z_ref[0], cbT_ref[...], (((0,), (0,)), ((), ())),
                           preferred_element_type=jnp.float32)
    scores = (a_ref[...] + b_ref[...]) - dot2
    colf = col_ref[...]

    def lexmin(s):
        m = jnp.min(s, axis=1, keepdims=True)
        gf = jnp.min(jnp.where(s == m, colf, jnp.float32(1 << 24)),
                     axis=1, keepdims=True)
        return m, gf.astype(jnp.int32)

    def merge_window(m, g):
        upd = m < wv[...]
        wv[...] = jnp.where(upd, m, wv[...])
        wi[...] = jnp.where(upd, g, wi[...])

    def close_window():
        take = wv[...] < cv[...]
        cv[...] = jnp.where(take, _bf16r(wv[...]), cv[...])
        ci[...] = jnp.where(take, wi[...], ci[...])
        vb[...] = jnp.where(take, wv[...], vb[...])

    @pl.when(jnp.logical_and(j != t1, j != t2))
    def _whole_tile():
        m, g = lexmin(scores)
        merge_window(m, g)

    def _split_tile(bound):
        cut = colf < jnp.float32(bound)
        seg1 = jnp.where(cut, scores, inf)
        m1, g1 = lexmin(seg1)
        merge_window(m1, g1)
        close_window()
        seg2 = jnp.where(cut, inf, scores)
        m2, g2 = lexmin(seg2)
        wv[...] = m2
        wi[...] = g2

    @pl.when(j == t1)
    def _boundary1():
        _split_tile((t1 * bn) + off1)

    @pl.when(j == t2)
    def _boundary2():
        _split_tile((t2 * bn) + off2)

    @pl.when(j == nj - 1)
    def _finish():
        take = wv[...] < cv[...]
        fidx = jnp.where(take, wi[...], ci[...])
        fval = jnp.where(take, wv[...], vb[...])
        idx_ref[...] = fidx

        @pl.when(i == 0)
        def _zero():
            acc[0] = 0.0

        acc[0] = acc[0] + jnp.sum(fval)

        @pl.when(i == pl.num_programs(0) - 1)
        def _emit():
            loss_ref[0, 0] = acc[0] * scale


def _distance_argmin(z16, cbT16, a, b2, colf, bm, bn):
    nb, e_dim, hw = z16.shape
    n_tok = nb * hw
    n_e = cbT16.shape[1]
    per_b = hw // bm
    grid = (n_tok // bm, n_e // bn)
    scale = (1.0 + _BETA) / (n_tok * e_dim)
    w = ((-(-n_e // 3) + 7) // 8) * 8
    t1, off1 = w // bn, w % bn
    t2, off2 = (2 * w) // bn, (2 * w) % bn
    statics = (grid[1], bn, scale, t1, off1, t2, off2)
    body = functools.partial(_argmin_body, statics)
    return pl.pallas_call(
        body,
        grid=grid,
        in_specs=[
            pl.BlockSpec((1, e_dim, bm),
                         lambda i, j: (i // per_b, 0, i % per_b)),
            pl.BlockSpec((e_dim, bn), lambda i, j: (0, j)),
            pl.BlockSpec((bm, 1), lambda i, j: (i, 0)),
            pl.BlockSpec((1, bn), lambda i, j: (0, j)),
            pl.BlockSpec((1, bn), lambda i, j: (0, j)),
        ],
        out_specs=[
            pl.BlockSpec((bm, 1), lambda i, j: (i, 0)),
            pl.BlockSpec(memory_space=pltpu.SMEM),
        ],
        out_shape=[
            jax.ShapeDtypeStruct((n_tok, 1), jnp.int32),
            jax.ShapeDtypeStruct((1, 1), jnp.float32),
        ],
        scratch_shapes=[
            pltpu.VMEM((bm, 1), jnp.float32),
            pltpu.VMEM((bm, 1), jnp.int32),
            pltpu.VMEM((bm, 1), jnp.float32),
            pltpu.VMEM((bm, 1), jnp.int32),
            pltpu.VMEM((bm, 1), jnp.float32),
            pltpu.SMEM((1,), jnp.float32),
        ],
        compiler_params=pltpu.CompilerParams(
            dimension_semantics=("arbitrary", "arbitrary")),
    )(z16, cbT16, a, b2, colf)


def _sc_gather(codebook, idx):
    n_e, e_dim = codebook.shape
    n_tok = idx.shape[0]
    info = plsc.get_sparse_core_info()
    nc, ns = info.num_cores, info.num_subcores
    nw = nc * ns
    b_per_w = n_tok // nw
    ch = 128
    n_ch = b_per_w // ch
    mesh = plsc.VectorSubcoreMesh(core_axis_name="c", subcore_axis_name="s")

    @functools.partial(
        pl.kernel, mesh=mesh,
        out_type=jax.ShapeDtypeStruct((n_tok, e_dim), jnp.float32),
        scratch_types=[
            pltpu.VMEM((b_per_w,), jnp.int32),
            pltpu.VMEM((ch, e_dim), jnp.float32),
            pltpu.VMEM((ch, e_dim), jnp.float32),
            pltpu.SemaphoreType.DMA,
            pltpu.SemaphoreType.DMA,
        ],
    )
    def gathered(table_hbm, idx_hbm, out_hbm, idx_v, rows_a, rows_b, sem_a,
                 sem_b):
        wid = lax.axis_index("s") * nc + lax.axis_index("c")
        base = wid * b_per_w
        pltpu.sync_copy(idx_hbm.at[pl.ds(base, b_per_w)], idx_v)
        bufs = (rows_a, rows_b)
        sems = (sem_a, sem_b)
        copies = []
        for c in range(n_ch):
            copies.append(pltpu.async_copy(
                table_hbm.at[idx_v.at[pl.ds(c * ch, ch)]],
                bufs[c % 2], sems[c % 2]))
            if c > 0:
                copies[c - 1].wait()
                pltpu.sync_copy(bufs[(c - 1) % 2],
                                out_hbm.at[pl.ds(base + (c - 1) * ch, ch)])
        copies[n_ch - 1].wait()
        pltpu.sync_copy(bufs[(n_ch - 1) % 2],
                        out_hbm.at[pl.ds(base + (n_ch - 1) * ch, ch)])

    return gathered(codebook, idx)


def kernel(z, codebook):
    B, C, H, W = z.shape
    n_e, e_dim = codebook.shape
    n_tok = B * H * W

    a = jnp.sum(z ** 2, axis=1).reshape(n_tok, 1)
    b2 = jnp.sum(codebook ** 2, axis=1).reshape(1, n_e)
    z16 = (z * 2.0).astype(jnp.bfloat16).reshape(B, C, H * W)
    cbT16 = codebook.T.astype(jnp.bfloat16)
    colf = jnp.arange(n_e, dtype=jnp.float32).reshape(1, n_e)

    bm = min(2048, H * W)
    bn = min(2048, n_e)
    idx2, loss11 = _distance_argmin(z16, cbT16, a, b2, colf, bm, bn)
    idx = idx2.reshape(n_tok)

    z_q = _sc_gather(codebook, idx)

    z_q_out = jnp.transpose(z_q.reshape(B, H, W, C), (0, 3, 1, 2))
    idx_out = idx.reshape(B, 1, H, W)
    return (z_q_out, loss11[0, 0], idx_out)
